# baseline (speedup 1.0000x reference)
import jax
import jax.numpy as jnp
from jax import lax
from jax.experimental import pallas as pl
from jax.experimental.pallas import tpu as pltpu
from jax.experimental.pallas import tpu_sc as plsc

N_NODES = 10000
N_EDGES = 320000
D_FEAT = 128

NC = 2
NS = 16
NW = NC * NS

E_PER_W = N_EDGES // NW
CHUNK = 96
NBUF = 4
N_CHUNKS = E_PER_W // CHUNK
TAIL = E_PER_W - N_CHUNKS * CHUNK

ROWS_PER_TILE = 624


def _gather_body(h_hbm, src_hbm, out_hbm, h_spmem, idx_bufs, row_bufs,
                 gat_sems, out_sems):
    cid = lax.axis_index("c")
    sid = lax.axis_index("s")
    wid = sid * NC + cid
    base = wid * E_PER_W

    stage_off = sid * ROWS_PER_TILE
    pltpu.sync_copy(h_hbm.at[pl.ds(stage_off, ROWS_PER_TILE)],
                    h_spmem.at[pl.ds(stage_off, ROWS_PER_TILE)])
    rem_off = NS * ROWS_PER_TILE
    rem = N_NODES - rem_off

    @pl.when(sid == NS - 1)
    def _stage_rem():
        pltpu.sync_copy(h_hbm.at[pl.ds(rem_off, rem)],
                        h_spmem.at[pl.ds(rem_off, rem)])

    plsc.subcore_barrier()

    def fill(b, j):
        pltpu.sync_copy(src_hbm.at[pl.ds(base + j * CHUNK, CHUNK)], idx_bufs[b])
        pltpu.async_copy(h_spmem.at[idx_bufs[b]], row_bufs[b], gat_sems[b])

    def wait_fill(b):
        pltpu.make_async_copy(
            h_spmem.at[idx_bufs[b]], row_bufs[b], gat_sems[b]
        ).wait()

    def scatter(b, j):
        pltpu.async_copy(
            row_bufs[b], out_hbm.at[pl.ds(base + j * CHUNK, CHUNK)], out_sems[b]
        )

    def wait_scatter(b, j):
        pltpu.make_async_copy(
            row_bufs[b], out_hbm.at[pl.ds(base + j * CHUNK, CHUNK)], out_sems[b]
        ).wait()

    for b in range(NBUF):
        fill(b, b)

    def group(gi, carry):
        g = gi * NBUF
        for b in range(NBUF):
            wait_fill(b)
            scatter(b, g + b)
        for b in range(NBUF):
            wait_scatter(b, g + b)
            fill(b, g + b + NBUF)
        return carry

    lax.fori_loop(0, N_CHUNKS // NBUF - 1, group, 0)

    last = N_CHUNKS - NBUF
    for b in range(NBUF):
        wait_fill(b)
        scatter(b, last + b)
    for b in range(NBUF):
        wait_scatter(b, last + b)

    toff = base + N_CHUNKS * CHUNK
    t_idx = idx_bufs[0].at[pl.ds(0, TAIL)]
    t_rows = row_bufs[0].at[pl.ds(0, TAIL)]
    pltpu.sync_copy(src_hbm.at[pl.ds(toff, TAIL)], t_idx)
    pltpu.async_copy(h_spmem.at[t_idx], t_rows, gat_sems[0])
    pltpu.make_async_copy(h_spmem.at[t_idx], t_rows, gat_sems[0]).wait()
    pltpu.sync_copy(t_rows, out_hbm.at[pl.ds(toff, TAIL)])


def _sc_gather(h, src):
    mesh = plsc.VectorSubcoreMesh(
        core_axis_name="c", subcore_axis_name="s", num_cores=NC, num_subcores=NS
    )
    scratch = (
        pltpu.VMEM_SHARED((N_NODES, D_FEAT), jnp.float32),
        [pltpu.VMEM((CHUNK,), jnp.int32) for _ in range(NBUF)],
        [pltpu.VMEM((CHUNK, D_FEAT), jnp.float32) for _ in range(NBUF)],
        [pltpu.SemaphoreType.DMA for _ in range(NBUF)],
        [pltpu.SemaphoreType.DMA for _ in range(NBUF)],
    )
    run = pl.kernel(
        _gather_body,
        out_type=jax.ShapeDtypeStruct((N_EDGES, D_FEAT), jnp.float32),
        mesh=mesh,
        scratch_types=scratch,
        name="sc_edge_gather",
    )
    return run(h, src)


@jax.jit
def kernel(h, edge_index):
    src = edge_index[0].astype(jnp.int32)
    return _sc_gather(h, src)


# speedup vs baseline: 1.0963x; 1.0963x over previous
import jax
import jax.numpy as jnp
from jax import lax
from jax.experimental import pallas as pl
from jax.experimental.pallas import tpu as pltpu
from jax.experimental.pallas import tpu_sc as plsc

N_NODES = 10000
N_EDGES = 320000
D_FEAT = 128

NC = 2
NS = 16
NW = NC * NS

E_PER_W = N_EDGES // NW
CHUNK = 128
NBUF = 3
N_CHUNKS = E_PER_W // CHUNK
TAIL = E_PER_W - N_CHUNKS * CHUNK


ROWS_PER_TILE = 624


def _gather_body(h_hbm, src_hbm, out_hbm, h_spmem, idx_bufs, row_bufs,
                 gat_sems, out_sems):
    cid = lax.axis_index("c")
    sid = lax.axis_index("s")
    wid = sid * NC + cid
    base = wid * E_PER_W

    stage_off = sid * ROWS_PER_TILE
    stage_len = jnp.where(sid == NS - 1, N_NODES - (NS - 1) * ROWS_PER_TILE,
                          ROWS_PER_TILE)
    pltpu.sync_copy(h_hbm.at[pl.ds(stage_off, ROWS_PER_TILE)],
                    h_spmem.at[pl.ds(stage_off, ROWS_PER_TILE)])
    del stage_len
    rem_off = NS * ROWS_PER_TILE
    rem = N_NODES - rem_off

    @pl.when(sid == NS - 1)
    def _stage_rem():
        pltpu.sync_copy(h_hbm.at[pl.ds(rem_off, rem)],
                        h_spmem.at[pl.ds(rem_off, rem)])

    plsc.subcore_barrier()

    def fill(b, j):
        pltpu.sync_copy(src_hbm.at[pl.ds(base + j * CHUNK, CHUNK)], idx_bufs[b])
        pltpu.async_copy(
            h_spmem.at[idx_bufs[b]], row_bufs[b], gat_sems[b]
        )

    def wait_fill(b, j):
        pltpu.make_async_copy(
            h_spmem.at[idx_bufs[b]], row_bufs[b], gat_sems[b]
        ).wait()

    def scatter(b, j):
        pltpu.async_copy(
            row_bufs[b], out_hbm.at[pl.ds(base + j * CHUNK, CHUNK)], out_sems[b]
        )

    def wait_scatter(b, j):
        pltpu.make_async_copy(
            row_bufs[b], out_hbm.at[pl.ds(base + j * CHUNK, CHUNK)], out_sems[b]
        ).wait()

    for b in range(NBUF):
        fill(b, b)

    def group(gi, carry):
        g = gi * NBUF
        for b in range(NBUF):
            wait_fill(b, g + b)
            scatter(b, g + b)
        for b in range(NBUF):
            wait_scatter(b, g + b)
            fill(b, g + b + NBUF)
        return carry

    lax.fori_loop(0, N_CHUNKS // NBUF - 1, group, 0)

    last = N_CHUNKS - NBUF
    for b in range(NBUF):
        wait_fill(b, last + b)
        scatter(b, last + b)
    for b in range(NBUF):
        wait_scatter(b, last + b)

    toff = base + N_CHUNKS * CHUNK
    t_idx = idx_bufs[0].at[pl.ds(0, TAIL)]
    t_rows = row_bufs[0].at[pl.ds(0, TAIL)]
    pltpu.sync_copy(src_hbm.at[pl.ds(toff, TAIL)], t_idx)
    pltpu.async_copy(h_spmem.at[t_idx], t_rows, gat_sems[0])
    pltpu.make_async_copy(h_spmem.at[t_idx], t_rows, gat_sems[0]).wait()
    pltpu.sync_copy(t_rows, out_hbm.at[pl.ds(toff, TAIL)])


def _sc_gather(h, src):
    mesh = plsc.VectorSubcoreMesh(
        core_axis_name="c", subcore_axis_name="s", num_cores=NC, num_subcores=NS
    )
    scratch = (
        pltpu.VMEM_SHARED((N_NODES, D_FEAT), jnp.float32),
        [pltpu.VMEM((CHUNK,), jnp.int32) for _ in range(NBUF)],
        [pltpu.VMEM((CHUNK, D_FEAT), jnp.float32) for _ in range(NBUF)],
        [pltpu.SemaphoreType.DMA for _ in range(NBUF)],
        [pltpu.SemaphoreType.DMA for _ in range(NBUF)],
    )
    run = pl.kernel(
        _gather_body,
        out_type=jax.ShapeDtypeStruct((N_EDGES, D_FEAT), jnp.float32),
        mesh=mesh,
        scratch_types=scratch,
        name="sc_edge_gather",
    )
    return run(h, src)


@jax.jit
def kernel(h, edge_index):
    src = edge_index[0].astype(jnp.int32)
    return _sc_gather(h, src)


# speedup vs baseline: 1.1028x; 1.0059x over previous
import jax
import jax.numpy as jnp
from jax import lax
from jax.experimental import pallas as pl
from jax.experimental.pallas import tpu as pltpu
from jax.experimental.pallas import tpu_sc as plsc

N_NODES = 10000
N_EDGES = 320000
D_FEAT = 128

NC = 2
NS = 16
NW = NC * NS

E_PER_W = N_EDGES // NW
CHUNK = 128
NBUF = 3
N_CHUNKS = E_PER_W // CHUNK
TAIL = E_PER_W - N_CHUNKS * CHUNK


ROWS_PER_TILE = 624


def _gather_body(h_hbm, src_hbm, out_hbm, h_spmem, idx_bufs, row_bufs,
                 gat_sems, out_sems, stage_sem):
    cid = lax.axis_index("c")
    sid = lax.axis_index("s")
    wid = sid * NC + cid
    base = wid * E_PER_W

    stage_off = sid * ROWS_PER_TILE
    pltpu.async_copy(h_hbm.at[pl.ds(stage_off, ROWS_PER_TILE)],
                     h_spmem.at[pl.ds(stage_off, ROWS_PER_TILE)], stage_sem)
    rem_off = NS * ROWS_PER_TILE
    rem = N_NODES - rem_off

    @pl.when(sid == NS - 1)
    def _stage_rem():
        pltpu.async_copy(h_hbm.at[pl.ds(rem_off, rem)],
                         h_spmem.at[pl.ds(rem_off, rem)], stage_sem)

    def fill(b, j):
        pltpu.sync_copy(src_hbm.at[pl.ds(base + j * CHUNK, CHUNK)], idx_bufs[b])
        pltpu.async_copy(
            h_spmem.at[idx_bufs[b]], row_bufs[b], gat_sems[b]
        )

    def wait_fill(b, j):
        pltpu.make_async_copy(
            h_spmem.at[idx_bufs[b]], row_bufs[b], gat_sems[b]
        ).wait()

    def scatter(b, j):
        pltpu.async_copy(
            row_bufs[b], out_hbm.at[pl.ds(base + j * CHUNK, CHUNK)], out_sems[b]
        )

    def wait_scatter(b, j):
        pltpu.make_async_copy(
            row_bufs[b], out_hbm.at[pl.ds(base + j * CHUNK, CHUNK)], out_sems[b]
        ).wait()

    for b in range(NBUF):
        pltpu.sync_copy(src_hbm.at[pl.ds(base + b * CHUNK, CHUNK)], idx_bufs[b])
        pltpu.async_copy(h_hbm.at[idx_bufs[b]], row_bufs[b], gat_sems[b])

    pltpu.make_async_copy(
        h_hbm.at[pl.ds(stage_off, ROWS_PER_TILE)],
        h_spmem.at[pl.ds(stage_off, ROWS_PER_TILE)], stage_sem
    ).wait()

    @pl.when(sid == NS - 1)
    def _wait_rem():
        pltpu.make_async_copy(h_hbm.at[pl.ds(rem_off, rem)],
                              h_spmem.at[pl.ds(rem_off, rem)], stage_sem).wait()

    plsc.subcore_barrier()

    def group(gi, carry):
        g = gi * NBUF
        for b in range(NBUF):
            wait_fill(b, g + b)
            scatter(b, g + b)
        for b in range(NBUF):
            wait_scatter(b, g + b)
            fill(b, g + b + NBUF)
        return carry

    lax.fori_loop(0, N_CHUNKS // NBUF - 1, group, 0)

    last = N_CHUNKS - NBUF
    for b in range(NBUF):
        wait_fill(b, last + b)
        scatter(b, last + b)
    for b in range(NBUF):
        wait_scatter(b, last + b)

    toff = base + N_CHUNKS * CHUNK
    t_idx = idx_bufs[0].at[pl.ds(0, TAIL)]
    t_rows = row_bufs[0].at[pl.ds(0, TAIL)]
    pltpu.sync_copy(src_hbm.at[pl.ds(toff, TAIL)], t_idx)
    pltpu.async_copy(h_spmem.at[t_idx], t_rows, gat_sems[0])
    pltpu.make_async_copy(h_spmem.at[t_idx], t_rows, gat_sems[0]).wait()
    pltpu.sync_copy(t_rows, out_hbm.at[pl.ds(toff, TAIL)])


def _sc_gather(h, src):
    mesh = plsc.VectorSubcoreMesh(
        core_axis_name="c", subcore_axis_name="s", num_cores=NC, num_subcores=NS
    )
    scratch = (
        pltpu.VMEM_SHARED((N_NODES, D_FEAT), jnp.float32),
        [pltpu.VMEM((CHUNK,), jnp.int32) for _ in range(NBUF)],
        [pltpu.VMEM((CHUNK, D_FEAT), jnp.float32) for _ in range(NBUF)],
        [pltpu.SemaphoreType.DMA for _ in range(NBUF)],
        [pltpu.SemaphoreType.DMA for _ in range(NBUF)],
        pltpu.SemaphoreType.DMA,
    )
    run = pl.kernel(
        _gather_body,
        out_type=jax.ShapeDtypeStruct((N_EDGES, D_FEAT), jnp.float32),
        mesh=mesh,
        scratch_types=scratch,
        name="sc_edge_gather",
    )
    return run(h, src)


@jax.jit
def kernel(h, edge_index):
    src = edge_index[0].astype(jnp.int32)
    return _sc_gather(h, src)
